# baseline (speedup 1.0000x reference)
import functools

import jax
import jax.numpy as jnp
from jax import lax
from jax.experimental import pallas as pl
from jax.experimental.pallas import tpu as pltpu
from jax.experimental.pallas import tpu_sc as plsc

_NW = 32
_TILE = 16384


def kernel(values, indices):
    n = values.shape[0]
    num_tiles = n // _TILE
    rem = n - num_tiles * _TILE
    mesh = plsc.VectorSubcoreMesh(core_axis_name="c", subcore_axis_name="s")

    @functools.partial(
        pl.kernel,
        mesh=mesh,
        out_type=jax.ShapeDtypeStruct((n,), jnp.float32),
        scratch_types=[
            pltpu.VMEM((_TILE,), jnp.float32),
            pltpu.VMEM((max(rem, 1),), jnp.float32),
        ],
    )
    def sc_copy(v_hbm, o_hbm, buf, tail_buf):
        cid = lax.axis_index("c")
        sid = lax.axis_index("s")
        wid = sid * 2 + cid

        my_tiles = jnp.where(num_tiles > wid, (num_tiles - 1 - wid) // _NW + 1, 0)

        def body(i, carry):
            off = (wid + i * _NW) * _TILE
            pltpu.sync_copy(v_hbm.at[pl.ds(off, _TILE)], buf)
            pltpu.sync_copy(buf, o_hbm.at[pl.ds(off, _TILE)])
            return carry

        lax.fori_loop(0, my_tiles, body, 0)

        if rem:
            @pl.when(wid == 0)
            def _tail():
                base = num_tiles * _TILE
                pltpu.sync_copy(v_hbm.at[pl.ds(base, rem)], tail_buf)
                pltpu.sync_copy(tail_buf, o_hbm.at[pl.ds(base, rem)])

    return sc_copy(values)


# speedup vs baseline: 3.4048x; 3.4048x over previous
import jax
import jax.numpy as jnp
from jax.experimental import pallas as pl

_GRID = 3


def _copy_block(v_ref, o_ref):
    o_ref[...] = v_ref[...]


def kernel(values, indices):
    n = values.shape[0]
    block = pl.cdiv(n, _GRID * 1024) * 1024
    grid = (pl.cdiv(n, block),)
    return pl.pallas_call(
        _copy_block,
        grid=grid,
        in_specs=[pl.BlockSpec((block,), lambda i: (i,))],
        out_specs=pl.BlockSpec((block,), lambda i: (i,)),
        out_shape=jax.ShapeDtypeStruct(values.shape, values.dtype),
    )(values)
